# Initial kernel scaffold; baseline (speedup 1.0000x reference)
import jax
import jax.numpy as jnp
from jax.experimental import pallas as pl


def kernel(x, edge_index, W_l1, b_l1, W_r1, W_l2, b_l2, W_r2):
    raise NotImplementedError("write your pallas kernel here")


# speedup vs baseline: 4.8412x; 4.8412x over previous
import jax
import jax.numpy as jnp
from jax import lax
from jax.experimental import pallas as pl
from jax.experimental.pallas import tpu as pltpu
from jax.experimental.pallas import tpu_sc as plsc

N = 10000
E = 320000
D = 128

_NC = 2
_NS = 16
_NW = _NC * _NS
_EPW = E // _NW
_C = 80
_CHUNKS = _EPW // _C
_NPAD = 10240
_RPT = _NPAD // _NS
_ZR = 128

_MESH = plsc.VectorSubcoreMesh(core_axis_name="c", subcore_axis_name="s")


def _sc_agg_body(x_hbm, src_hbm, dst_hbm, agg_hbm,
                 srcv, dstv, rows, zbufD, aggs, sem):
    cid = lax.axis_index("c")
    sid = lax.axis_index("s")
    wid = sid * _NC + cid

    def fill_zD(r, carry):
        for j in range(D // 16):
            zbufD[r, pl.ds(j * 16, 16)] = jnp.zeros((16,), jnp.float32)
        return carry

    lax.fori_loop(0, _ZR, fill_zD, 0)

    r0 = pl.multiple_of(sid * _RPT, 8)
    for k in range(_RPT // _ZR):
        pltpu.sync_copy(zbufD, aggs.at[pl.ds(r0 + k * _ZR, _ZR)])
    plsc.subcore_barrier()

    ebase = wid * _EPW

    def step(i, carry):
        base = pl.multiple_of(ebase + i * _C, 8)
        pltpu.sync_copy(src_hbm.at[pl.ds(base, _C)], srcv)
        pltpu.sync_copy(dst_hbm.at[pl.ds(base, _C)], dstv)
        pltpu.async_copy(x_hbm.at[srcv], rows, sem).wait()
        pltpu.sync_copy(rows, aggs.at[dstv], add=True)
        return carry

    lax.fori_loop(0, _CHUNKS, step, 0)
    plsc.subcore_barrier()

    pltpu.sync_copy(aggs.at[pl.ds(r0, 400)], agg_hbm.at[cid, pl.ds(r0, 400)])

    @pl.when(sid < _NS - 1)
    def _tail():
        r1 = pl.multiple_of(r0 + 400, 8)
        pltpu.sync_copy(aggs.at[pl.ds(r1, 240)],
                        agg_hbm.at[cid, pl.ds(r1, 240)])


_sc_agg = pl.kernel(
    _sc_agg_body,
    out_type=jax.ShapeDtypeStruct((_NC, N, D), jnp.float32),
    mesh=_MESH,
    scratch_types=[
        pltpu.VMEM((_C,), jnp.int32),
        pltpu.VMEM((_C,), jnp.int32),
        pltpu.VMEM((_C, D), jnp.float32),
        pltpu.VMEM((_ZR, D), jnp.float32),
        pltpu.VMEM_SHARED((_NPAD, D), jnp.float32),
        pltpu.SemaphoreType.DMA,
    ],
)


def _sc_cnt_body(dst_hbm, cnt_hbm, dstv, onesv, zbufD, cnts):
    cid = lax.axis_index("c")
    sid = lax.axis_index("s")
    wid = sid * _NC + cid

    def fill_zD(r, carry):
        for j in range(D // 16):
            zbufD[r, pl.ds(j * 16, 16)] = jnp.zeros((16,), jnp.float32)
        return carry

    lax.fori_loop(0, _ZR, fill_zD, 0)

    def fill_ones(r, carry):
        for j in range(D // 16):
            onesv[r, pl.ds(j * 16, 16)] = jnp.ones((16,), jnp.float32)
        return carry

    lax.fori_loop(0, _C, fill_ones, 0)

    r0 = pl.multiple_of(sid * _RPT, 8)
    for k in range(_RPT // _ZR):
        pltpu.sync_copy(zbufD, cnts.at[pl.ds(r0 + k * _ZR, _ZR)])
    plsc.subcore_barrier()

    ebase = wid * _EPW

    def step(i, carry):
        base = pl.multiple_of(ebase + i * _C, 8)
        pltpu.sync_copy(dst_hbm.at[pl.ds(base, _C)], dstv)
        pltpu.sync_copy(onesv, cnts.at[dstv], add=True)
        return carry

    lax.fori_loop(0, _CHUNKS, step, 0)
    plsc.subcore_barrier()

    pltpu.sync_copy(cnts.at[pl.ds(r0, 400)], cnt_hbm.at[cid, pl.ds(r0, 400)])

    @pl.when(sid < _NS - 1)
    def _tail():
        r1 = pl.multiple_of(r0 + 400, 8)
        pltpu.sync_copy(cnts.at[pl.ds(r1, 240)],
                        cnt_hbm.at[cid, pl.ds(r1, 240)])


_sc_cnt = pl.kernel(
    _sc_cnt_body,
    out_type=jax.ShapeDtypeStruct((_NC, N, D), jnp.float32),
    mesh=_MESH,
    scratch_types=[
        pltpu.VMEM((_C,), jnp.int32),
        pltpu.VMEM((_C, D), jnp.float32),
        pltpu.VMEM((_ZR, D), jnp.float32),
        pltpu.VMEM_SHARED((_NPAD, D), jnp.float32),
    ],
)

_R = 1000


def _tc_dense(aggp, cntb, x, wl, b, wr, relu):
    def body(aggp_ref, cnt_ref, x_ref, wl_ref, b_ref, wr_ref, o_ref):
        agg = aggp_ref[0] + aggp_ref[1]
        agg = agg / jnp.maximum(cnt_ref[...], 1.0)
        h = lax.dot_general(agg, wl_ref[...], (((1,), (1,)), ((), ())),
                            preferred_element_type=jnp.float32)
        h = h + b_ref[...]
        h = h + lax.dot_general(x_ref[...], wr_ref[...], (((1,), (1,)), ((), ())),
                                preferred_element_type=jnp.float32)
        if relu:
            h = jnp.maximum(h, 0.0)
        o_ref[...] = h

    return pl.pallas_call(
        body,
        grid=(N // _R,),
        in_specs=[
            pl.BlockSpec((_NC, _R, D), lambda i: (0, i, 0)),
            pl.BlockSpec((_R, D), lambda i: (i, 0)),
            pl.BlockSpec((_R, D), lambda i: (i, 0)),
            pl.BlockSpec((D, D), lambda i: (0, 0)),
            pl.BlockSpec((1, D), lambda i: (0, 0)),
            pl.BlockSpec((D, D), lambda i: (0, 0)),
        ],
        out_specs=pl.BlockSpec((_R, D), lambda i: (i, 0)),
        out_shape=jax.ShapeDtypeStruct((N, D), jnp.float32),
    )(aggp, cntb, x, wl, b, wr)


def kernel(x, edge_index, W_l1, b_l1, W_r1, W_l2, b_l2, W_r2):
    src = edge_index[0]
    dst = edge_index[1]
    cntp = _sc_cnt(dst)
    cnt = cntp[0, :, 0] + cntp[1, :, 0]
    cntb = jnp.broadcast_to(cnt[:, None], (N, D))

    aggp1 = _sc_agg(x, src, dst)
    h = _tc_dense(aggp1, cntb, x, W_l1, b_l1.reshape(1, D), W_r1, relu=True)
    aggp2 = _sc_agg(h, src, dst)
    out = _tc_dense(aggp2, cntb, h, W_l2, b_l2.reshape(1, D), W_r2, relu=False)
    return out


# speedup vs baseline: 7.7397x; 1.5987x over previous
import jax
import jax.numpy as jnp
from jax import lax
from jax.experimental import pallas as pl
from jax.experimental.pallas import tpu as pltpu
from jax.experimental.pallas import tpu_sc as plsc

N = 10000
E = 320000
D = 128

_NC = 2
_NS = 16
_NW = _NC * _NS
_EPW = E // _NW
_C = 80
_CHUNKS = _EPW // _C
_NBUF = 5
_ROUNDS = _CHUNKS // _NBUF
_NPAD = 10240
_RPT = _NPAD // _NS
_ZR = 128

_MESH = plsc.VectorSubcoreMesh(core_axis_name="c", subcore_axis_name="s")


def _sc_agg_body(x_hbm, edge_hbm, agg_hbm,
                 rows0, rows1, ibuf0, ibuf1, dstv0, dstv1,
                 isem0, isem1, gsem0, gsem1, ssem0, ssem1, aggs):
    rows = (rows0, rows1)
    ibuf = (ibuf0, ibuf1)
    dstv = (dstv0, dstv1)
    isem = (isem0, isem1)
    gsem = (gsem0, gsem1)
    ssem = (ssem0, ssem1)

    cid = lax.axis_index("c")
    sid = lax.axis_index("s")
    wid = sid * _NC + cid

    def fill_z(r, carry):
        for j in range(D // 16):
            rows0[r, pl.ds(j * 16, 16)] = jnp.zeros((16,), jnp.float32)
        return carry

    lax.fori_loop(0, _C, fill_z, 0)
    r0 = pl.multiple_of(sid * _RPT, 8)
    for k in range(_RPT // _C):
        pltpu.sync_copy(rows0, aggs.at[pl.ds(r0 + k * _C, _C)])
    plsc.subcore_barrier()

    ebase = pl.multiple_of(wid * _CHUNKS * 2 * _C, 8)

    def issue_idx(i, b):
        pltpu.async_copy(edge_hbm.at[pl.ds(ebase + i * 2 * _C, 2 * _C)],
                         ibuf[b], isem[b])

    def wait_idx(b):
        pltpu.make_async_copy(edge_hbm.at[pl.ds(0, 2 * _C)],
                              ibuf[b], isem[b]).wait()

    def issue_gather(i, b):
        del i
        pltpu.async_copy(x_hbm.at[ibuf[b].at[pl.ds(0, _C)]], rows[b], gsem[b])

    def wait_gather(b):
        pltpu.make_async_copy(x_hbm.at[pl.ds(0, _C)], rows[b], gsem[b]).wait()

    def issue_scatter(b):
        for j in range(_C // 16):
            dstv[b][pl.ds(j * 16, 16)] = ibuf[b][pl.ds(_C + j * 16, 16)]
        pltpu.async_copy(rows[b], aggs.at[dstv[b]], ssem[b], add=True)

    def wait_scatter(b):
        pltpu.make_async_copy(x_hbm.at[pl.ds(0, _C)], rows[b], ssem[b]).wait()

    issue_idx(0, 0)
    issue_idx(1, 1)
    wait_idx(0)
    issue_gather(0, 0)
    wait_gather(0)
    issue_scatter(0)
    issue_idx(2, 0)
    wait_idx(1)
    issue_gather(1, 1)

    def pair(g, carry):
        for b, off in ((1, 1), (0, 2)):
            i = g * 2 + off
            wait_gather(b)
            issue_scatter(b)
            issue_idx(i + 2, b)
            wait_scatter(1 - b)
            wait_idx(1 - b)
            issue_gather(i + 1, 1 - b)
        return carry

    lax.fori_loop(0, (_CHUNKS - 3) // 2, pair, 0)

    wait_gather(1)
    issue_scatter(1)
    wait_scatter(0)
    wait_idx(0)
    issue_gather(_CHUNKS - 1, 0)
    wait_gather(0)
    issue_scatter(0)
    wait_scatter(1)
    wait_scatter(0)
    plsc.subcore_barrier()

    pltpu.sync_copy(aggs.at[pl.ds(r0, 400)], agg_hbm.at[cid, pl.ds(r0, 400)])

    @pl.when(sid < _NS - 1)
    def _tail():
        r1 = pl.multiple_of(r0 + 400, 8)
        pltpu.sync_copy(aggs.at[pl.ds(r1, 240)],
                        agg_hbm.at[cid, pl.ds(r1, 240)])


_sc_agg = pl.kernel(
    _sc_agg_body,
    out_type=jax.ShapeDtypeStruct((_NC, N, D), jnp.float32),
    mesh=_MESH,
    scratch_types=(
        [pltpu.VMEM((_C, D), jnp.float32) for _ in range(2)]
        + [pltpu.VMEM((2 * _C,), jnp.int32) for _ in range(2)]
        + [pltpu.VMEM((_C,), jnp.int32) for _ in range(2)]
        + [pltpu.SemaphoreType.DMA for _ in range(6)]
        + [pltpu.VMEM_SHARED((_NPAD, D), jnp.float32)]
    ),
)


def _sc_cnt_body(dst_hbm, cnt_hbm, dstv, onesv, zbufD, cnts):
    cid = lax.axis_index("c")
    sid = lax.axis_index("s")
    wid = sid * _NC + cid

    def fill_zD(r, carry):
        for j in range(D // 16):
            zbufD[r, pl.ds(j * 16, 16)] = jnp.zeros((16,), jnp.float32)
        return carry

    lax.fori_loop(0, _ZR, fill_zD, 0)

    def fill_ones(r, carry):
        for j in range(D // 16):
            onesv[r, pl.ds(j * 16, 16)] = jnp.ones((16,), jnp.float32)
        return carry

    lax.fori_loop(0, _C, fill_ones, 0)

    r0 = pl.multiple_of(sid * _RPT, 8)
    for k in range(_RPT // _ZR):
        pltpu.sync_copy(zbufD, cnts.at[pl.ds(r0 + k * _ZR, _ZR)])
    plsc.subcore_barrier()

    ebase = wid * _EPW

    def step(i, carry):
        base = pl.multiple_of(ebase + i * _C, 8)
        pltpu.sync_copy(dst_hbm.at[pl.ds(base, _C)], dstv)
        pltpu.sync_copy(onesv, cnts.at[dstv], add=True)
        return carry

    lax.fori_loop(0, _CHUNKS, step, 0)
    plsc.subcore_barrier()

    pltpu.sync_copy(cnts.at[pl.ds(r0, 400)], cnt_hbm.at[cid, pl.ds(r0, 400)])

    @pl.when(sid < _NS - 1)
    def _tail():
        r1 = pl.multiple_of(r0 + 400, 8)
        pltpu.sync_copy(cnts.at[pl.ds(r1, 240)],
                        cnt_hbm.at[cid, pl.ds(r1, 240)])


_sc_cnt = pl.kernel(
    _sc_cnt_body,
    out_type=jax.ShapeDtypeStruct((_NC, N, D), jnp.float32),
    mesh=_MESH,
    scratch_types=[
        pltpu.VMEM((_C,), jnp.int32),
        pltpu.VMEM((_C, D), jnp.float32),
        pltpu.VMEM((_ZR, D), jnp.float32),
        pltpu.VMEM_SHARED((_NPAD, D), jnp.float32),
    ],
)

_R = 1000


def _tc_dense(aggp, cntp, x, wl, b, wr, relu):
    def body(aggp_ref, cntp_ref, x_ref, wl_ref, b_ref, wr_ref, o_ref):
        agg = aggp_ref[0] + aggp_ref[1]
        cnt = cntp_ref[0] + cntp_ref[1]
        agg = agg / jnp.maximum(cnt, 1.0)
        h = lax.dot_general(agg, wl_ref[...], (((1,), (1,)), ((), ())),
                            preferred_element_type=jnp.float32)
        h = h + b_ref[...]
        h = h + lax.dot_general(x_ref[...], wr_ref[...], (((1,), (1,)), ((), ())),
                                preferred_element_type=jnp.float32)
        if relu:
            h = jnp.maximum(h, 0.0)
        o_ref[...] = h

    return pl.pallas_call(
        body,
        grid=(N // _R,),
        in_specs=[
            pl.BlockSpec((_NC, _R, D), lambda i: (0, i, 0)),
            pl.BlockSpec((_NC, _R, D), lambda i: (0, i, 0)),
            pl.BlockSpec((_R, D), lambda i: (i, 0)),
            pl.BlockSpec((D, D), lambda i: (0, 0)),
            pl.BlockSpec((1, D), lambda i: (0, 0)),
            pl.BlockSpec((D, D), lambda i: (0, 0)),
        ],
        out_specs=pl.BlockSpec((_R, D), lambda i: (i, 0)),
        out_shape=jax.ShapeDtypeStruct((N, D), jnp.float32),
    )(aggp, cntp, x, wl, b, wr)


def kernel(x, edge_index, W_l1, b_l1, W_r1, W_l2, b_l2, W_r2):
    dst = edge_index[1]
    cntp = _sc_cnt(dst)
    il = jnp.concatenate(
        [edge_index[0].reshape(E // _C, _C), edge_index[1].reshape(E // _C, _C)],
        axis=1).reshape(-1)
    aggp1 = _sc_agg(x, il)
    h = _tc_dense(aggp1, cntp, x, W_l1, b_l1.reshape(1, D), W_r1, relu=True)
    aggp2 = _sc_agg(h, il)
    out = _tc_dense(aggp2, cntp, h, W_l2, b_l2.reshape(1, D), W_r2, relu=False)
    return out


# speedup vs baseline: 10.0309x; 1.2960x over previous
import jax
import jax.numpy as jnp
from jax import lax
from jax.experimental import pallas as pl
from jax.experimental.pallas import tpu as pltpu
from jax.experimental.pallas import tpu_sc as plsc

N = 10000
E = 320000
D = 128

_NC = 2
_NS = 16
_NW = _NC * _NS
_EPW = E // _NW
_C = 80
_CHUNKS = _EPW // _C
_NBUF = 5
_ROUNDS = _CHUNKS // _NBUF
_NPAD = 10240
_RPT = _NPAD // _NS
_ZR = 128

_MESH = plsc.VectorSubcoreMesh(core_axis_name="c", subcore_axis_name="s")


def _sc_agg_body(x_hbm, edge_hbm, agg_hbm, cnt_hbm,
                 rows0, rows1, ibuf0, ibuf1, dstv0, dstv1, onesv, zbuf1,
                 isem0, isem1, gsem0, gsem1, ssem0, ssem1, csem0, csem1,
                 aggs, cnts):
    rows = (rows0, rows1)
    ibuf = (ibuf0, ibuf1)
    dstv = (dstv0, dstv1)
    isem = (isem0, isem1)
    gsem = (gsem0, gsem1)
    ssem = (ssem0, ssem1)
    csem = (csem0, csem1)

    cid = lax.axis_index("c")
    sid = lax.axis_index("s")
    wid = sid * _NC + cid

    def fill_z(r, carry):
        for j in range(D // 16):
            rows0[r, pl.ds(j * 16, 16)] = jnp.zeros((16,), jnp.float32)
        return carry

    lax.fori_loop(0, _C, fill_z, 0)

    def fill_ones(r, carry):
        onesv[pl.ds(r * 16, 16)] = jnp.ones((16,), jnp.float32)
        return carry

    lax.fori_loop(0, _C // 16, fill_ones, 0)

    def fill_z1(r, carry):
        zbuf1[pl.ds(r * 16, 16)] = jnp.zeros((16,), jnp.float32)
        return carry

    lax.fori_loop(0, _RPT // 16, fill_z1, 0)

    r0 = pl.multiple_of(sid * _RPT, 8)
    for k in range(_RPT // _C):
        pltpu.sync_copy(rows0, aggs.at[pl.ds(r0 + k * _C, _C)])
    pltpu.sync_copy(zbuf1, cnts.at[pl.ds(r0, _RPT)])
    plsc.subcore_barrier()

    ebase = pl.multiple_of(wid * _CHUNKS * 2 * _C, 8)

    def issue_idx(i, b):
        pltpu.async_copy(edge_hbm.at[pl.ds(ebase + i * 2 * _C, 2 * _C)],
                         ibuf[b], isem[b])

    def wait_idx(b):
        pltpu.make_async_copy(edge_hbm.at[pl.ds(0, 2 * _C)],
                              ibuf[b], isem[b]).wait()

    def issue_gather(i, b):
        del i
        pltpu.async_copy(x_hbm.at[ibuf[b].at[pl.ds(0, _C)]], rows[b], gsem[b])

    def wait_gather(b):
        pltpu.make_async_copy(x_hbm.at[pl.ds(0, _C)], rows[b], gsem[b]).wait()

    def issue_scatter(b):
        for j in range(_C // 16):
            dstv[b][pl.ds(j * 16, 16)] = ibuf[b][pl.ds(_C + j * 16, 16)]
        pltpu.async_copy(rows[b], aggs.at[dstv[b]], ssem[b], add=True)
        pltpu.async_copy(onesv, cnts.at[dstv[b]], csem[b], add=True)

    def wait_scatter(b):
        pltpu.make_async_copy(x_hbm.at[pl.ds(0, _C)], rows[b], ssem[b]).wait()
        pltpu.make_async_copy(edge_hbm.at[pl.ds(0, _C)], dstv[b],
                              csem[b]).wait()

    issue_idx(0, 0)
    issue_idx(1, 1)
    wait_idx(0)
    issue_gather(0, 0)
    wait_gather(0)
    issue_scatter(0)
    issue_idx(2, 0)
    wait_idx(1)
    issue_gather(1, 1)

    def pair(g, carry):
        for b, off in ((1, 1), (0, 2)):
            i = g * 2 + off
            wait_gather(b)
            issue_scatter(b)
            issue_idx(i + 2, b)
            wait_scatter(1 - b)
            wait_idx(1 - b)
            issue_gather(i + 1, 1 - b)
        return carry

    lax.fori_loop(0, (_CHUNKS - 3) // 2, pair, 0)

    wait_gather(1)
    issue_scatter(1)
    wait_scatter(0)
    wait_idx(0)
    issue_gather(_CHUNKS - 1, 0)
    wait_gather(0)
    issue_scatter(0)
    wait_scatter(1)
    wait_scatter(0)
    plsc.subcore_barrier()

    pltpu.sync_copy(aggs.at[pl.ds(r0, 400)], agg_hbm.at[cid, pl.ds(r0, 400)])
    pltpu.sync_copy(cnts.at[pl.ds(r0, _RPT)], cnt_hbm.at[cid, pl.ds(r0, _RPT)])

    @pl.when(sid < _NS - 1)
    def _tail():
        r1 = pl.multiple_of(r0 + 400, 8)
        pltpu.sync_copy(aggs.at[pl.ds(r1, 240)],
                        agg_hbm.at[cid, pl.ds(r1, 240)])


_sc_agg = pl.kernel(
    _sc_agg_body,
    out_type=(jax.ShapeDtypeStruct((_NC, N, D), jnp.float32),
              jax.ShapeDtypeStruct((_NC, _NPAD), jnp.float32)),
    mesh=_MESH,
    scratch_types=(
        [pltpu.VMEM((_C, D), jnp.float32) for _ in range(2)]
        + [pltpu.VMEM((2 * _C,), jnp.int32) for _ in range(2)]
        + [pltpu.VMEM((_C,), jnp.int32) for _ in range(2)]
        + [pltpu.VMEM((_C,), jnp.float32),
           pltpu.VMEM((_RPT,), jnp.float32)]
        + [pltpu.SemaphoreType.DMA for _ in range(8)]
        + [pltpu.VMEM_SHARED((_NPAD, D), jnp.float32),
           pltpu.VMEM_SHARED((_NPAD,), jnp.float32)]
    ),
)

_R = 1000


def _tc_dense(aggp, cntb, x, wl, b, wr, relu):
    def body(aggp_ref, cnt_ref, x_ref, wl_ref, b_ref, wr_ref, o_ref):
        agg = aggp_ref[0] + aggp_ref[1]
        agg = agg / jnp.maximum(cnt_ref[...], 1.0)
        h = lax.dot_general(agg, wl_ref[...], (((1,), (1,)), ((), ())),
                            preferred_element_type=jnp.float32)
        h = h + b_ref[...]
        h = h + lax.dot_general(x_ref[...], wr_ref[...], (((1,), (1,)), ((), ())),
                                preferred_element_type=jnp.float32)
        if relu:
            h = jnp.maximum(h, 0.0)
        o_ref[...] = h

    return pl.pallas_call(
        body,
        grid=(N // _R,),
        in_specs=[
            pl.BlockSpec((_NC, _R, D), lambda i: (0, i, 0)),
            pl.BlockSpec((_R, D), lambda i: (i, 0)),
            pl.BlockSpec((_R, D), lambda i: (i, 0)),
            pl.BlockSpec((D, D), lambda i: (0, 0)),
            pl.BlockSpec((1, D), lambda i: (0, 0)),
            pl.BlockSpec((D, D), lambda i: (0, 0)),
        ],
        out_specs=pl.BlockSpec((_R, D), lambda i: (i, 0)),
        out_shape=jax.ShapeDtypeStruct((N, D), jnp.float32),
    )(aggp, cntb, x, wl, b, wr)


def kernel(x, edge_index, W_l1, b_l1, W_r1, W_l2, b_l2, W_r2):
    il = jnp.concatenate(
        [edge_index[0].reshape(E // _C, _C), edge_index[1].reshape(E // _C, _C)],
        axis=1).reshape(-1)
    aggp1, cntp = _sc_agg(x, il)
    cntb = jnp.broadcast_to((cntp[0, :N] + cntp[1, :N])[:, None], (N, D))
    h = _tc_dense(aggp1, cntb, x, W_l1, b_l1.reshape(1, D), W_r1, relu=True)
    aggp2, _ = _sc_agg(h, il)
    out = _tc_dense(aggp2, cntb, h, W_l2, b_l2.reshape(1, D), W_r2, relu=False)
    return out


# speedup vs baseline: 10.9039x; 1.0870x over previous
import jax
import jax.numpy as jnp
from jax import lax
from jax.experimental import pallas as pl
from jax.experimental.pallas import tpu as pltpu
from jax.experimental.pallas import tpu_sc as plsc

N = 10000
E = 320000
D = 128

_NC = 2
_NS = 16
_NW = _NC * _NS
_EPW = E // _NW
_C = 40
_CHUNKS = _EPW // _C
_NBUF = 5
_ROUNDS = _CHUNKS // _NBUF
_NPAD = 10240
_RPT = _NPAD // _NS
_ZR = 128

_MESH = plsc.VectorSubcoreMesh(core_axis_name="c", subcore_axis_name="s")

_OFFS = sorted({min(j * 16, _C - 16) for j in range((_C + 15) // 16)})


def _sc_agg_body(x_hbm, edge_hbm, agg_hbm, cnt_hbm, *rest):
    rows = rest[0:_NBUF]
    ibuf = rest[_NBUF:2 * _NBUF]
    dstv = rest[2 * _NBUF:3 * _NBUF]
    onesv = rest[3 * _NBUF]
    zbuf1 = rest[3 * _NBUF + 1]
    isem = rest[3 * _NBUF + 2:4 * _NBUF + 2]
    gsem = rest[4 * _NBUF + 2:5 * _NBUF + 2]
    ssem = rest[5 * _NBUF + 2:6 * _NBUF + 2]
    csem = rest[6 * _NBUF + 2:7 * _NBUF + 2]
    aggs = rest[7 * _NBUF + 2]
    cnts = rest[7 * _NBUF + 3]
    rows0 = rows[0]

    cid = lax.axis_index("c")
    sid = lax.axis_index("s")
    wid = sid * _NC + cid

    def fill_z(r, carry):
        for j in range(D // 16):
            rows0[r, pl.ds(j * 16, 16)] = jnp.zeros((16,), jnp.float32)
        return carry

    lax.fori_loop(0, _C, fill_z, 0)

    for o in _OFFS:
        onesv[pl.ds(o, 16)] = jnp.ones((16,), jnp.float32)

    def fill_z1(r, carry):
        zbuf1[pl.ds(r * 16, 16)] = jnp.zeros((16,), jnp.float32)
        return carry

    lax.fori_loop(0, _RPT // 16, fill_z1, 0)

    r0 = pl.multiple_of(sid * _RPT, 8)
    for k in range(_RPT // _C):
        pltpu.sync_copy(rows0, aggs.at[pl.ds(r0 + k * _C, _C)])
    pltpu.sync_copy(zbuf1, cnts.at[pl.ds(r0, _RPT)])
    plsc.subcore_barrier()

    ebase = pl.multiple_of(wid * _CHUNKS * 2 * _C, 8)

    def issue_idx(i, b):
        pltpu.async_copy(edge_hbm.at[pl.ds(ebase + i * 2 * _C, 2 * _C)],
                         ibuf[b], isem[b])

    def wait_idx(b):
        pltpu.make_async_copy(edge_hbm.at[pl.ds(0, 2 * _C)],
                              ibuf[b], isem[b]).wait()

    def issue_gather(i, b):
        del i
        pltpu.async_copy(x_hbm.at[ibuf[b].at[pl.ds(0, _C)]], rows[b], gsem[b])

    def wait_gather(b):
        pltpu.make_async_copy(x_hbm.at[pl.ds(0, _C)], rows[b], gsem[b]).wait()

    def issue_scatter(b):
        for o in _OFFS:
            dstv[b][pl.ds(o, 16)] = ibuf[b][pl.ds(_C + o, 16)]
        pltpu.async_copy(rows[b], aggs.at[dstv[b]], ssem[b], add=True)
        pltpu.async_copy(onesv, cnts.at[dstv[b]], csem[b], add=True)

    def wait_scatter(b):
        pltpu.make_async_copy(x_hbm.at[pl.ds(0, _C)], rows[b], ssem[b]).wait()
        pltpu.make_async_copy(edge_hbm.at[pl.ds(0, _C)], dstv[b],
                              csem[b]).wait()

    for b in range(_NBUF):
        issue_idx(b, b)
    for b in range(_NBUF):
        wait_idx(b)
        issue_gather(b, b)

    def round_fn(g, carry):
        for b in range(_NBUF):
            wait_gather(b)
            issue_scatter(b)

        @pl.when(g < _ROUNDS - 1)
        def _refill():
            for b in range(_NBUF):
                issue_idx(g * _NBUF + b + _NBUF, b)
            for b in range(_NBUF):
                wait_scatter(b)
                wait_idx(b)
                issue_gather(g * _NBUF + b + _NBUF, b)

        return carry

    lax.fori_loop(0, _ROUNDS, round_fn, 0)

    for b in range(_NBUF):
        wait_scatter(b)
    plsc.subcore_barrier()

    pltpu.sync_copy(aggs.at[pl.ds(r0, 400)], agg_hbm.at[cid, pl.ds(r0, 400)])
    pltpu.sync_copy(cnts.at[pl.ds(r0, _RPT)], cnt_hbm.at[cid, pl.ds(r0, _RPT)])

    @pl.when(sid < _NS - 1)
    def _tail():
        r1 = pl.multiple_of(r0 + 400, 8)
        pltpu.sync_copy(aggs.at[pl.ds(r1, 240)],
                        agg_hbm.at[cid, pl.ds(r1, 240)])


_sc_agg = pl.kernel(
    _sc_agg_body,
    out_type=(jax.ShapeDtypeStruct((_NC, N, D), jnp.float32),
              jax.ShapeDtypeStruct((_NC, _NPAD), jnp.float32)),
    mesh=_MESH,
    scratch_types=(
        [pltpu.VMEM((_C, D), jnp.float32) for _ in range(_NBUF)]
        + [pltpu.VMEM((2 * _C,), jnp.int32) for _ in range(_NBUF)]
        + [pltpu.VMEM((_C,), jnp.int32) for _ in range(_NBUF)]
        + [pltpu.VMEM((_C,), jnp.float32),
           pltpu.VMEM((_RPT,), jnp.float32)]
        + [pltpu.SemaphoreType.DMA for _ in range(4 * _NBUF)]
        + [pltpu.VMEM_SHARED((_NPAD, D), jnp.float32),
           pltpu.VMEM_SHARED((_NPAD,), jnp.float32)]
    ),
)

_R = 1000


def _tc_dense(aggp, cntb, x, wl, b, wr, relu):
    def body(aggp_ref, cnt_ref, x_ref, wl_ref, b_ref, wr_ref, o_ref):
        agg = aggp_ref[0] + aggp_ref[1]
        agg = agg / jnp.maximum(cnt_ref[...], 1.0)
        h = lax.dot_general(agg, wl_ref[...], (((1,), (1,)), ((), ())),
                            preferred_element_type=jnp.float32)
        h = h + b_ref[...]
        h = h + lax.dot_general(x_ref[...], wr_ref[...], (((1,), (1,)), ((), ())),
                                preferred_element_type=jnp.float32)
        if relu:
            h = jnp.maximum(h, 0.0)
        o_ref[...] = h

    return pl.pallas_call(
        body,
        grid=(N // _R,),
        in_specs=[
            pl.BlockSpec((_NC, _R, D), lambda i: (0, i, 0)),
            pl.BlockSpec((_R, D), lambda i: (i, 0)),
            pl.BlockSpec((_R, D), lambda i: (i, 0)),
            pl.BlockSpec((D, D), lambda i: (0, 0)),
            pl.BlockSpec((1, D), lambda i: (0, 0)),
            pl.BlockSpec((D, D), lambda i: (0, 0)),
        ],
        out_specs=pl.BlockSpec((_R, D), lambda i: (i, 0)),
        out_shape=jax.ShapeDtypeStruct((N, D), jnp.float32),
    )(aggp, cntb, x, wl, b, wr)


def kernel(x, edge_index, W_l1, b_l1, W_r1, W_l2, b_l2, W_r2):
    il = jnp.concatenate(
        [edge_index[0].reshape(E // _C, _C), edge_index[1].reshape(E // _C, _C)],
        axis=1).reshape(-1)
    aggp1, cntp = _sc_agg(x, il)
    cntb = jnp.broadcast_to((cntp[0, :N] + cntp[1, :N])[:, None], (N, D))
    h = _tc_dense(aggp1, cntb, x, W_l1, b_l1.reshape(1, D), W_r1, relu=True)
    aggp2, _ = _sc_agg(h, il)
    out = _tc_dense(aggp2, cntb, h, W_l2, b_l2.reshape(1, D), W_r2, relu=False)
    return out


# speedup vs baseline: 11.0561x; 1.0140x over previous
import jax
import jax.numpy as jnp
from jax import lax
from jax.experimental import pallas as pl
from jax.experimental.pallas import tpu as pltpu
from jax.experimental.pallas import tpu_sc as plsc

N = 10000
E = 320000
D = 128

_NC = 2
_NS = 16
_NW = _NC * _NS
_EPW = E // _NW
_C = 40
_CHUNKS = _EPW // _C
_NBUF = 5
_ROUNDS = _CHUNKS // _NBUF
_NPAD = 10240
_RPT = _NPAD // _NS
_ZR = 128

_MESH = plsc.VectorSubcoreMesh(core_axis_name="c", subcore_axis_name="s")

_OFFS = sorted({min(j * 16, _C - 16) for j in range((_C + 15) // 16)})


def _sc_agg_body(x_hbm, edge_hbm, agg_hbm, cnt_hbm, *rest):
    rows = rest[0:_NBUF]
    ibuf = rest[_NBUF:2 * _NBUF]
    dstv = rest[2 * _NBUF:3 * _NBUF]
    onesv = rest[3 * _NBUF]
    zbuf1 = rest[3 * _NBUF + 1]
    isem = rest[3 * _NBUF + 2:4 * _NBUF + 2]
    gsem = rest[4 * _NBUF + 2:5 * _NBUF + 2]
    ssem = rest[5 * _NBUF + 2:6 * _NBUF + 2]
    csem = rest[6 * _NBUF + 2:7 * _NBUF + 2]
    aggs = rest[7 * _NBUF + 2]
    cnts = rest[7 * _NBUF + 3]
    rows0 = rows[0]

    cid = lax.axis_index("c")
    sid = lax.axis_index("s")
    wid = sid * _NC + cid

    def fill_z(r, carry):
        for j in range(D // 16):
            rows0[r, pl.ds(j * 16, 16)] = jnp.zeros((16,), jnp.float32)
        return carry

    lax.fori_loop(0, _C, fill_z, 0)

    for o in _OFFS:
        onesv[pl.ds(o, 16)] = jnp.ones((16,), jnp.float32)

    def fill_z1(r, carry):
        zbuf1[pl.ds(r * 16, 16)] = jnp.zeros((16,), jnp.float32)
        return carry

    lax.fori_loop(0, _RPT // 16, fill_z1, 0)

    ebase = pl.multiple_of(wid * _CHUNKS * 2 * _C, 8)
    r0 = pl.multiple_of(sid * _RPT, 8)

    def issue_idx(i, b):
        pltpu.async_copy(edge_hbm.at[pl.ds(ebase + i * 2 * _C, 2 * _C)],
                         ibuf[b], isem[b])

    def wait_idx(b):
        pltpu.make_async_copy(edge_hbm.at[pl.ds(0, 2 * _C)],
                              ibuf[b], isem[b]).wait()

    def issue_gather(i, b):
        del i
        pltpu.async_copy(x_hbm.at[ibuf[b].at[pl.ds(0, _C)]], rows[b], gsem[b])

    def wait_gather(b):
        pltpu.make_async_copy(x_hbm.at[pl.ds(0, _C)], rows[b], gsem[b]).wait()

    def issue_scatter(b):
        for o in _OFFS:
            dstv[b][pl.ds(o, 16)] = ibuf[b][pl.ds(_C + o, 16)]
        pltpu.async_copy(rows[b], aggs.at[dstv[b]], ssem[b], add=True)
        pltpu.async_copy(onesv, cnts.at[dstv[b]], csem[b], add=True)

    def wait_scatter(b):
        pltpu.make_async_copy(x_hbm.at[pl.ds(0, _C)], rows[b], ssem[b]).wait()
        pltpu.make_async_copy(edge_hbm.at[pl.ds(0, _C)], dstv[b],
                              csem[b]).wait()

    for b in range(_NBUF):
        issue_idx(b, b)
    for k in range(_RPT // _C):
        pltpu.async_copy(rows0, aggs.at[pl.ds(r0 + k * _C, _C)], csem[0])
    pltpu.sync_copy(zbuf1, cnts.at[pl.ds(r0, _RPT)])
    for b in range(1, _NBUF):
        wait_idx(b)
        issue_gather(b, b)
    for k in range(_RPT // _C):
        pltpu.make_async_copy(x_hbm.at[pl.ds(0, _C)], rows0, csem[0]).wait()
    plsc.subcore_barrier()
    wait_idx(0)
    issue_gather(0, 0)

    def round_fn(g, carry):
        for b in range(_NBUF):
            wait_gather(b)
            issue_scatter(b)

        @pl.when(g < _ROUNDS - 1)
        def _refill():
            for b in range(_NBUF):
                issue_idx(g * _NBUF + b + _NBUF, b)
            for b in range(_NBUF):
                wait_scatter(b)
                wait_idx(b)
                issue_gather(g * _NBUF + b + _NBUF, b)

        return carry

    lax.fori_loop(0, _ROUNDS, round_fn, 0)

    for b in range(_NBUF):
        wait_scatter(b)
    plsc.subcore_barrier()

    d1 = pltpu.async_copy(aggs.at[pl.ds(r0, 400)],
                          agg_hbm.at[cid, pl.ds(r0, 400)], isem[0])
    d2 = pltpu.async_copy(cnts.at[pl.ds(r0, _RPT)],
                          cnt_hbm.at[cid, pl.ds(r0, _RPT)], isem[1])

    @pl.when(sid < _NS - 1)
    def _tail():
        r1 = pl.multiple_of(r0 + 400, 8)
        pltpu.sync_copy(aggs.at[pl.ds(r1, 240)],
                        agg_hbm.at[cid, pl.ds(r1, 240)])

    d1.wait()
    d2.wait()


_sc_agg = pl.kernel(
    _sc_agg_body,
    out_type=(jax.ShapeDtypeStruct((_NC, N, D), jnp.float32),
              jax.ShapeDtypeStruct((_NC, _NPAD), jnp.float32)),
    mesh=_MESH,
    scratch_types=(
        [pltpu.VMEM((_C, D), jnp.float32) for _ in range(_NBUF)]
        + [pltpu.VMEM((2 * _C,), jnp.int32) for _ in range(_NBUF)]
        + [pltpu.VMEM((_C,), jnp.int32) for _ in range(_NBUF)]
        + [pltpu.VMEM((_C,), jnp.float32),
           pltpu.VMEM((_RPT,), jnp.float32)]
        + [pltpu.SemaphoreType.DMA for _ in range(4 * _NBUF)]
        + [pltpu.VMEM_SHARED((_NPAD, D), jnp.float32),
           pltpu.VMEM_SHARED((_NPAD,), jnp.float32)]
    ),
)

_R = 1000


def _tc_dense(aggp, cntb, x, wl, b, wr, relu):
    def body(aggp_ref, cnt_ref, x_ref, wl_ref, b_ref, wr_ref, o_ref):
        agg = aggp_ref[0] + aggp_ref[1]
        agg = agg / jnp.maximum(cnt_ref[...], 1.0)
        h = lax.dot_general(agg, wl_ref[...], (((1,), (1,)), ((), ())),
                            preferred_element_type=jnp.float32)
        h = h + b_ref[...]
        h = h + lax.dot_general(x_ref[...], wr_ref[...], (((1,), (1,)), ((), ())),
                                preferred_element_type=jnp.float32)
        if relu:
            h = jnp.maximum(h, 0.0)
        o_ref[...] = h

    return pl.pallas_call(
        body,
        grid=(N // _R,),
        in_specs=[
            pl.BlockSpec((_NC, _R, D), lambda i: (0, i, 0)),
            pl.BlockSpec((_R, D), lambda i: (i, 0)),
            pl.BlockSpec((_R, D), lambda i: (i, 0)),
            pl.BlockSpec((D, D), lambda i: (0, 0)),
            pl.BlockSpec((1, D), lambda i: (0, 0)),
            pl.BlockSpec((D, D), lambda i: (0, 0)),
        ],
        out_specs=pl.BlockSpec((_R, D), lambda i: (i, 0)),
        out_shape=jax.ShapeDtypeStruct((N, D), jnp.float32),
    )(aggp, cntb, x, wl, b, wr)


def kernel(x, edge_index, W_l1, b_l1, W_r1, W_l2, b_l2, W_r2):
    il = jnp.concatenate(
        [edge_index[0].reshape(E // _C, _C), edge_index[1].reshape(E // _C, _C)],
        axis=1).reshape(-1)
    aggp1, cntp = _sc_agg(x, il)
    cntb = jnp.broadcast_to((cntp[0, :N] + cntp[1, :N])[:, None], (N, D))
    h = _tc_dense(aggp1, cntb, x, W_l1, b_l1.reshape(1, D), W_r1, relu=True)
    aggp2, _ = _sc_agg(h, il)
    out = _tc_dense(aggp2, cntb, h, W_l2, b_l2.reshape(1, D), W_r2, relu=False)
    return out


# speedup vs baseline: 11.6520x; 1.0539x over previous
import jax
import jax.numpy as jnp
from jax import lax
from jax.experimental import pallas as pl
from jax.experimental.pallas import tpu as pltpu
from jax.experimental.pallas import tpu_sc as plsc

N = 10000
E = 320000
D = 128

_NC = 2
_NS = 16
_NW = _NC * _NS
_EPW = E // _NW
_C = 80
_CHUNKS = _EPW // _C
_NBUF = 3
_ROUNDS = _CHUNKS // _NBUF
_LEFT = _CHUNKS - _ROUNDS * _NBUF
_NPAD = 10112
_RPT = _NPAD // _NS

_MESH = plsc.VectorSubcoreMesh(core_axis_name="c", subcore_axis_name="s")

def _windows(n):
    return sorted({min(j * 16, n - 16) for j in range((n + 15) // 16)})


_OFFS = _windows(_C)


def _sc_agg_body(x_hbm, edge_hbm, agg_hbm, cnt_hbm, *rest):
    rows = rest[0:_NBUF]
    ibuf = rest[_NBUF:2 * _NBUF]
    dstv = rest[2 * _NBUF:3 * _NBUF]
    onesv = rest[3 * _NBUF]
    zbuf1 = rest[3 * _NBUF + 1]
    isem = rest[3 * _NBUF + 2:4 * _NBUF + 2]
    gsem = rest[4 * _NBUF + 2:5 * _NBUF + 2]
    ssem = rest[5 * _NBUF + 2:6 * _NBUF + 2]
    csem = rest[6 * _NBUF + 2:7 * _NBUF + 2]
    aggs = rest[7 * _NBUF + 2]
    cnts = rest[7 * _NBUF + 3]
    rows0 = rows[0]

    cid = lax.axis_index("c")
    sid = lax.axis_index("s")
    wid = sid * _NC + cid

    def fill_z(r, carry):
        for j in range(D // 16):
            rows0[r, pl.ds(j * 16, 16)] = jnp.zeros((16,), jnp.float32)
        return carry

    lax.fori_loop(0, _C, fill_z, 0)

    for o in _OFFS:
        onesv[pl.ds(o, 16)] = jnp.ones((16,), jnp.float32)

    for o in _windows(_RPT):
        zbuf1[pl.ds(o, 16)] = jnp.zeros((16,), jnp.float32)

    ebase = pl.multiple_of(wid * _CHUNKS * 2 * _C, 8)
    r0 = pl.multiple_of(sid * _RPT, 8)

    def issue_idx(i, b):
        pltpu.async_copy(edge_hbm.at[pl.ds(ebase + i * 2 * _C, 2 * _C)],
                         ibuf[b], isem[b])

    def wait_idx(b):
        pltpu.make_async_copy(edge_hbm.at[pl.ds(0, 2 * _C)],
                              ibuf[b], isem[b]).wait()

    def issue_gather(i, b):
        del i
        pltpu.async_copy(x_hbm.at[ibuf[b].at[pl.ds(0, _C)]], rows[b], gsem[b])

    def wait_gather(b):
        pltpu.make_async_copy(x_hbm.at[pl.ds(0, _C)], rows[b], gsem[b]).wait()

    def issue_scatter(b):
        for o in _OFFS:
            dstv[b][pl.ds(o, 16)] = ibuf[b][pl.ds(_C + o, 16)]
        pltpu.async_copy(rows[b], aggs.at[dstv[b]], ssem[b], add=True)
        pltpu.async_copy(onesv, cnts.at[dstv[b]], csem[b], add=True)

    def wait_scatter(b):
        pltpu.make_async_copy(x_hbm.at[pl.ds(0, _C)], rows[b], ssem[b]).wait()
        pltpu.make_async_copy(edge_hbm.at[pl.ds(0, _C)], dstv[b],
                              csem[b]).wait()

    for b in range(_NBUF):
        issue_idx(b, b)
    _ZTAIL = _RPT - (_RPT // _C) * _C
    for k in range(_RPT // _C):
        pltpu.async_copy(rows0, aggs.at[pl.ds(r0 + k * _C, _C)], csem[0])
    pltpu.async_copy(rows0.at[pl.ds(0, _ZTAIL)],
                     aggs.at[pl.ds(r0 + (_RPT // _C) * _C, _ZTAIL)], csem[0])
    pltpu.sync_copy(zbuf1, cnts.at[pl.ds(r0, _RPT)])
    for b in range(1, _NBUF):
        wait_idx(b)
        issue_gather(b, b)
    for k in range(_RPT // _C):
        pltpu.make_async_copy(x_hbm.at[pl.ds(0, _C)], rows0, csem[0]).wait()
    pltpu.make_async_copy(x_hbm.at[pl.ds(0, _ZTAIL)],
                          rows0.at[pl.ds(0, _ZTAIL)], csem[0]).wait()
    plsc.subcore_barrier()
    wait_idx(0)
    issue_gather(0, 0)

    def round_fn(g, carry):
        for b in range(_NBUF):
            wait_gather(b)
            issue_scatter(b)

        @pl.when(g < _ROUNDS - 1)
        def _refill():
            for b in range(_NBUF):
                issue_idx(g * _NBUF + b + _NBUF, b)
            for b in range(_NBUF):
                wait_scatter(b)
                wait_idx(b)
                issue_gather(g * _NBUF + b + _NBUF, b)

        return carry

    lax.fori_loop(0, _ROUNDS, round_fn, 0)

    for b in range(_LEFT):
        issue_idx(_ROUNDS * _NBUF + b, b)
    for b in range(_LEFT):
        wait_scatter(b)
        wait_idx(b)
        issue_gather(_ROUNDS * _NBUF + b, b)
    for b in range(_LEFT):
        wait_gather(b)
        issue_scatter(b)
    for b in range(_NBUF):
        wait_scatter(b)
    plsc.subcore_barrier()

    d1 = pltpu.async_copy(aggs.at[pl.ds(r0, 520)],
                          agg_hbm.at[cid, pl.ds(r0, 520)], isem[0])
    r0c = pl.multiple_of(sid * 640, 8)
    d2 = pltpu.async_copy(cnts.at[pl.ds(r0c, 640)],
                          cnt_hbm.at[cid, pl.ds(r0c, 640)], isem[1])

    @pl.when(sid < _NS - 1)
    def _tail():
        r1 = pl.multiple_of(r0 + 520, 8)
        pltpu.sync_copy(aggs.at[pl.ds(r1, 112)],
                        agg_hbm.at[cid, pl.ds(r1, 112)])

    d1.wait()
    d2.wait()


_sc_agg = pl.kernel(
    _sc_agg_body,
    out_type=(jax.ShapeDtypeStruct((_NC, N, D), jnp.float32),
              jax.ShapeDtypeStruct((_NC, 10240), jnp.float32)),
    mesh=_MESH,
    scratch_types=(
        [pltpu.VMEM((_C, D), jnp.float32) for _ in range(_NBUF)]
        + [pltpu.VMEM((2 * _C,), jnp.int32) for _ in range(_NBUF)]
        + [pltpu.VMEM((_C,), jnp.int32) for _ in range(_NBUF)]
        + [pltpu.VMEM((_C,), jnp.float32),
           pltpu.VMEM((_RPT,), jnp.float32)]
        + [pltpu.SemaphoreType.DMA for _ in range(4 * _NBUF)]
        + [pltpu.VMEM_SHARED((_NPAD, D), jnp.float32),
           pltpu.VMEM_SHARED((10240,), jnp.float32)]
    ),
)

_R = 1000


def _tc_dense(aggp, cntb, x, wl, b, wr, relu):
    def body(aggp_ref, cnt_ref, x_ref, wl_ref, b_ref, wr_ref, o_ref):
        agg = aggp_ref[0] + aggp_ref[1]
        agg = agg / jnp.maximum(cnt_ref[...], 1.0)
        h = lax.dot_general(agg, wl_ref[...], (((1,), (1,)), ((), ())),
                            preferred_element_type=jnp.float32)
        h = h + b_ref[...]
        h = h + lax.dot_general(x_ref[...], wr_ref[...], (((1,), (1,)), ((), ())),
                                preferred_element_type=jnp.float32)
        if relu:
            h = jnp.maximum(h, 0.0)
        o_ref[...] = h

    return pl.pallas_call(
        body,
        grid=(N // _R,),
        in_specs=[
            pl.BlockSpec((_NC, _R, D), lambda i: (0, i, 0)),
            pl.BlockSpec((_R, D), lambda i: (i, 0)),
            pl.BlockSpec((_R, D), lambda i: (i, 0)),
            pl.BlockSpec((D, D), lambda i: (0, 0)),
            pl.BlockSpec((1, D), lambda i: (0, 0)),
            pl.BlockSpec((D, D), lambda i: (0, 0)),
        ],
        out_specs=pl.BlockSpec((_R, D), lambda i: (i, 0)),
        out_shape=jax.ShapeDtypeStruct((N, D), jnp.float32),
    )(aggp, cntb, x, wl, b, wr)


def kernel(x, edge_index, W_l1, b_l1, W_r1, W_l2, b_l2, W_r2):
    il = jnp.concatenate(
        [edge_index[0].reshape(E // _C, _C), edge_index[1].reshape(E // _C, _C)],
        axis=1).reshape(-1)
    aggp1, cntp = _sc_agg(x, il)
    cntb = jnp.broadcast_to((cntp[0, :N] + cntp[1, :N])[:, None], (N, D))
    h = _tc_dense(aggp1, cntb, x, W_l1, b_l1.reshape(1, D), W_r1, relu=True)
    aggp2, _ = _sc_agg(h, il)
    out = _tc_dense(aggp2, cntb, h, W_l2, b_l2.reshape(1, D), W_r2, relu=False)
    return out
